# baseline (speedup 1.0000x reference)
import functools

import jax
import jax.numpy as jnp
from jax.experimental import pallas as pl
from jax.experimental.pallas import tpu as pltpu


def _chamfer_body(x1_ref, x2_ref, out_ref, d2min_ref, *, ni_blocks, inv_n):
    b = pl.program_id(0)
    i = pl.program_id(1)

    x1 = x1_ref[0]
    x2 = x2_ref[0]

    sq1 = jnp.sum(x1 * x1, axis=0, keepdims=True)
    sq2 = jnp.sum(x2 * x2, axis=0, keepdims=True)

    inner2 = jax.lax.dot_general(
        x1 * -2.0, x2, (((0,), (0,)), ((), ())),
        preferred_element_type=jnp.float32,
    )
    e = (inner2 + sq1.T) + sq2

    row_min = jnp.min(e, axis=1, keepdims=True)
    s1 = jnp.sum(row_min)

    col_min = jnp.min(e, axis=0, keepdims=True)

    @pl.when(i == 0)
    def _init():
        d2min_ref[...] = col_min

    @pl.when(i > 0)
    def _acc():
        d2min_ref[...] = jnp.minimum(d2min_ref[...], col_min)

    @pl.when(jnp.logical_and(b == 0, i == 0))
    def _zero():
        out_ref[0, 0] = 0.0

    out_ref[0, 0] += s1 * inv_n

    @pl.when(i == ni_blocks - 1)
    def _flush():
        out_ref[0, 0] += jnp.sum(d2min_ref[...]) * inv_n


def kernel(xyz1, xyz2):
    B, N1, _ = xyz1.shape
    _, N2, _ = xyz2.shape
    BI = 512
    ni_blocks = N1 // BI

    x1t = jnp.transpose(xyz1, (0, 2, 1))
    x2t = jnp.transpose(xyz2, (0, 2, 1))

    body = functools.partial(
        _chamfer_body, ni_blocks=ni_blocks, inv_n=1.0 / float(B * N1)
    )

    out = pl.pallas_call(
        body,
        grid=(B, ni_blocks),
        in_specs=[
            pl.BlockSpec((1, 3, BI), lambda b, i: (b, 0, i)),
            pl.BlockSpec((1, 3, N2), lambda b, i: (b, 0, 0)),
        ],
        out_specs=pl.BlockSpec(
            (1, 1), lambda b, i: (0, 0), memory_space=pltpu.SMEM
        ),
        out_shape=jax.ShapeDtypeStruct((1, 1), jnp.float32),
        scratch_shapes=[pltpu.VMEM((1, N2), jnp.float32)],
    )(x1t, x2t)
    return out[0, 0]


# speedup vs baseline: 1.1825x; 1.1825x over previous
import functools

import jax
import jax.numpy as jnp
from jax.experimental import pallas as pl
from jax.experimental.pallas import tpu as pltpu


def _split3(v):
    a = v.astype(jnp.bfloat16)
    r = v - a.astype(jnp.float32)
    b = r.astype(jnp.bfloat16)
    c = (r - b.astype(jnp.float32)).astype(jnp.bfloat16)
    return a, b, c


def _chamfer_body(x1_ref, x2_ref, out_ref, aug2_ref, d2min_ref, *, ni_blocks, inv_n):
    b = pl.program_id(0)
    i = pl.program_id(1)

    x1 = x1_ref[0]
    bi = x1.shape[1]

    @pl.when(i == 0)
    def _build_aug2():
        x2 = x2_ref[0]
        n2 = x2.shape[1]
        x2h = x2.astype(jnp.bfloat16)
        x2l = (x2 - x2h.astype(jnp.float32)).astype(jnp.bfloat16)
        sq2 = jnp.sum(x2 * x2, axis=0, keepdims=True)
        s2h, s2m, s2l = _split3(sq2)
        one = jnp.ones((1, n2), jnp.bfloat16)
        zero = jnp.zeros((1, n2), jnp.bfloat16)
        aug2_ref[...] = jnp.concatenate(
            [x2h, x2l, x2h, one, one, one, s2h, s2m, s2l, zero], axis=0
        )

    x1m = x1 * -2.0
    x1h = x1m.astype(jnp.bfloat16)
    x1l = (x1m - x1h.astype(jnp.float32)).astype(jnp.bfloat16)
    sq1 = jnp.sum(x1 * x1, axis=0, keepdims=True)
    s1h, s1m_, s1l = _split3(sq1)
    one1 = jnp.ones((1, bi), jnp.bfloat16)
    zero1 = jnp.zeros((1, bi), jnp.bfloat16)
    aug1 = jnp.concatenate(
        [x1h, x1h, x1l, s1h, s1m_, s1l, one1, one1, one1, zero1], axis=0
    )

    e = jax.lax.dot_general(
        aug1, aug2_ref[...], (((0,), (0,)), ((), ())),
        preferred_element_type=jnp.float32,
    )

    row_min = jnp.min(e, axis=1, keepdims=True)
    s1_sum = jnp.sum(row_min)

    col_min = jnp.min(e, axis=0, keepdims=True)

    @pl.when(i == 0)
    def _init():
        d2min_ref[...] = col_min

    @pl.when(i > 0)
    def _acc():
        d2min_ref[...] = jnp.minimum(d2min_ref[...], col_min)

    @pl.when(jnp.logical_and(b == 0, i == 0))
    def _zero():
        out_ref[0, 0] = 0.0

    out_ref[0, 0] += s1_sum * inv_n

    @pl.when(i == ni_blocks - 1)
    def _flush():
        out_ref[0, 0] += jnp.sum(d2min_ref[...]) * inv_n


def kernel(xyz1, xyz2):
    B, N1, _ = xyz1.shape
    _, N2, _ = xyz2.shape
    BI = 512
    ni_blocks = N1 // BI

    x1t = jnp.transpose(xyz1, (0, 2, 1))
    x2t = jnp.transpose(xyz2, (0, 2, 1))

    body = functools.partial(
        _chamfer_body, ni_blocks=ni_blocks, inv_n=1.0 / float(B * N1)
    )

    out = pl.pallas_call(
        body,
        grid=(B, ni_blocks),
        in_specs=[
            pl.BlockSpec((1, 3, BI), lambda b, i: (b, 0, i)),
            pl.BlockSpec((1, 3, N2), lambda b, i: (b, 0, 0)),
        ],
        out_specs=pl.BlockSpec(
            (1, 1), lambda b, i: (0, 0), memory_space=pltpu.SMEM
        ),
        out_shape=jax.ShapeDtypeStruct((1, 1), jnp.float32),
        scratch_shapes=[
            pltpu.VMEM((16, N2), jnp.bfloat16),
            pltpu.VMEM((1, N2), jnp.float32),
        ],
    )(x1t, x2t)
    return out[0, 0]
